# Initial kernel scaffold; baseline (speedup 1.0000x reference)
import jax
import jax.numpy as jnp
from jax.experimental import pallas as pl


def kernel(input_ids, emb_weight):
    raise NotImplementedError("write your pallas kernel here")


# speedup vs baseline: 1.0288x; 1.0288x over previous
import functools

import jax
import jax.numpy as jnp
from jax import lax
from jax.experimental import pallas as pl
from jax.experimental.pallas import tpu as pltpu
from jax.experimental.pallas import tpu_sc as plsc

NC = 2
NS = 16
NW = NC * NS
CHUNK = 128
DPAD = 384


@functools.lru_cache(maxsize=None)
def _make_lookup(n_tokens: int):
    assert n_tokens % (NW * CHUNK) == 0
    b_per_w = n_tokens // NW
    n_chunks = b_per_w // CHUNK
    mesh = plsc.VectorSubcoreMesh(core_axis_name="c", subcore_axis_name="s")

    @functools.partial(
        pl.kernel,
        mesh=mesh,
        out_type=jax.ShapeDtypeStruct((n_tokens, DPAD), jnp.float32),
        scratch_types=[
            pltpu.VMEM((n_chunks, CHUNK), jnp.int32),
            pltpu.VMEM((CHUNK, DPAD), jnp.float32),
            pltpu.SemaphoreType.DMA,
        ],
    )
    def lookup(idx_hbm, table_hbm, out_hbm, idx_v, rows_v, g_sem):
        wid = lax.axis_index("s") * NC + lax.axis_index("c")
        base = wid * b_per_w
        pltpu.sync_copy(idx_hbm.at[wid], idx_v)

        def body(g, carry):
            pltpu.async_copy(table_hbm.at[idx_v.at[g]], rows_v, g_sem).wait()
            pltpu.sync_copy(rows_v, out_hbm.at[pl.ds(base + g * CHUNK, CHUNK)])
            return carry

        lax.fori_loop(0, n_chunks, body, 0)

    return lookup


def kernel(input_ids, emb_weight):
    b, l = input_ids.shape
    vocab, dim = emb_weight.shape
    n = b * l
    idx = input_ids.reshape(NW, n // (NW * CHUNK), CHUNK).astype(jnp.int32)
    table = jnp.pad(emb_weight, ((0, 0), (0, DPAD - dim)))
    out = _make_lookup(n)(idx, table)
    return out[:, :dim].reshape(b, l, dim)
